# Initial kernel scaffold; baseline (speedup 1.0000x reference)
import jax
import jax.numpy as jnp
from jax.experimental import pallas as pl


def kernel(x, pos_table):
    raise NotImplementedError("write your pallas kernel here")


# speedup vs baseline: 2.8225x; 2.8225x over previous
import jax
import jax.numpy as jnp
from jax.experimental import pallas as pl


def _add_kernel(x_ref, p_ref, o_ref):
    o_ref[...] = x_ref[...] + p_ref[...]


def kernel(x, pos_table):
    b, s, h = x.shape
    bs = 512
    grid = (s // bs, b)
    return pl.pallas_call(
        _add_kernel,
        grid=grid,
        in_specs=[
            pl.BlockSpec((1, bs, h), lambda i, j: (j, i, 0)),
            pl.BlockSpec((bs, h), lambda i, j: (i, 0)),
        ],
        out_specs=pl.BlockSpec((1, bs, h), lambda i, j: (j, i, 0)),
        out_shape=jax.ShapeDtypeStruct((b, s, h), x.dtype),
    )(x, pos_table)
